# Initial kernel scaffold; baseline (speedup 1.0000x reference)
import jax
import jax.numpy as jnp
from jax.experimental import pallas as pl


def kernel(x, edge_index, eps0, eps1, W0a, b0a, W0b, b0b, W1a, b1a, W1b, b1b):
    raise NotImplementedError("write your pallas kernel here")


# speedup vs baseline: 7.0943x; 7.0943x over previous
import functools

import jax
import jax.numpy as jnp
from jax import lax
from jax.experimental import pallas as pl
from jax.experimental.pallas import tpu as pltpu
from jax.experimental.pallas import tpu_sc as plsc

NC = 2
NS = 16
C = 125


def _segment_sum_sc(table, src2, dst2, n_rows):
    e_chunks, _ = src2.shape
    d = table.shape[1]
    chunks_per_tile = e_chunks // (NC * NS)
    n_pad = ((n_rows + NS * 8 - 1) // (NS * 8)) * (NS * 8)
    rows_per_tile = n_pad // NS
    zr = 8
    mesh = plsc.VectorSubcoreMesh(core_axis_name="c", subcore_axis_name="s")

    @functools.partial(
        pl.kernel,
        out_type=jax.ShapeDtypeStruct((NC, n_pad, d), jnp.float32),
        mesh=mesh,
        scratch_types=[
            pltpu.VMEM((chunks_per_tile, C), jnp.int32),
            pltpu.VMEM((chunks_per_tile, C), jnp.int32),
            pltpu.VMEM((C, d), jnp.float32),
            pltpu.VMEM((zr, d), jnp.float32),
            pltpu.VMEM_SHARED((n_pad, d), jnp.float32),
            pltpu.SemaphoreType.DMA,
        ],
    )
    def seg_sum(table_hbm, src_hbm, dst_hbm, out_hbm,
                src_v, dst_v, rows_v, zbuf, acc, sem):
        cid = lax.axis_index("c")
        sid = lax.axis_index("s")
        tile = cid * NS + sid

        pltpu.sync_copy(src_hbm.at[pl.ds(tile * chunks_per_tile, chunks_per_tile)], src_v)
        pltpu.sync_copy(dst_hbm.at[pl.ds(tile * chunks_per_tile, chunks_per_tile)], dst_v)

        zero16 = jnp.zeros((16,), jnp.float32)

        def zrow(r, carry):
            for j in range(d // 16):
                zbuf[r, pl.ds(j * 16, 16)] = zero16
            return carry

        lax.fori_loop(0, zr, zrow, 0)
        row0 = sid * rows_per_tile
        for k in range(rows_per_tile // zr):
            pltpu.sync_copy(zbuf, acc.at[pl.ds(row0 + k * zr, zr)])
        plsc.subcore_barrier()

        def body(i, carry):
            pltpu.async_copy(table_hbm.at[src_v.at[i]], rows_v, sem).wait()
            pltpu.sync_copy(rows_v, acc.at[dst_v.at[i]], add=True)
            return carry

        lax.fori_loop(0, chunks_per_tile, body, 0)
        plsc.subcore_barrier()

        pltpu.sync_copy(acc.at[pl.ds(row0, rows_per_tile)],
                        out_hbm.at[cid, pl.ds(row0, rows_per_tile)])

    return seg_sum(table, src2, dst2)[:, :n_rows, :]


def _mlp0_tc(x, p0, p1, eps, Wa, ba, Wb, bb, block_rows=1000):
    n, din = x.shape
    h = Wa.shape[1]

    def body(eps_ref, x_ref, p0_ref, p1_ref, wa_ref, ba_ref, wb_ref, bb_ref, o_ref):
        t = (1.0 + eps_ref[0, 0]) * x_ref[...] + p0_ref[...] + p1_ref[...]
        t = jnp.dot(t, wa_ref[...], preferred_element_type=jnp.float32) + ba_ref[...]
        t = jnp.maximum(t, 0.0)
        t = jnp.dot(t, wb_ref[...], preferred_element_type=jnp.float32) + bb_ref[...]
        t = jnp.maximum(t, 0.0)
        o_ref[...] = jnp.concatenate(
            [t, jnp.zeros((t.shape[0], t.shape[1]), jnp.float32)], axis=1)

    grid = (n // block_rows,)
    return pl.pallas_call(
        body,
        grid=grid,
        in_specs=[
            pl.BlockSpec(memory_space=pltpu.SMEM),
            pl.BlockSpec((block_rows, din), lambda i: (i, 0)),
            pl.BlockSpec((block_rows, din), lambda i: (i, 0)),
            pl.BlockSpec((block_rows, din), lambda i: (i, 0)),
            pl.BlockSpec(Wa.shape, lambda i: (0, 0)),
            pl.BlockSpec(ba.shape, lambda i: (0, 0)),
            pl.BlockSpec(Wb.shape, lambda i: (0, 0)),
            pl.BlockSpec(bb.shape, lambda i: (0, 0)),
        ],
        out_specs=pl.BlockSpec((block_rows, 2 * h), lambda i: (i, 0)),
        out_shape=jax.ShapeDtypeStruct((n, 2 * h), jnp.float32),
    )(eps, x, p0, p1, Wa, ba, Wb, bb)


def _mlp1_tc(hin, p0, p1, eps, Wa, ba, Wb, bb, block_rows=1000):
    n, h = hin.shape
    dout = Wb.shape[1]

    def body(eps_ref, h_ref, p0_ref, p1_ref, wa_ref, ba_ref, wb_ref, bb_ref, o_ref):
        t = (1.0 + eps_ref[0, 0]) * h_ref[...] + p0_ref[...] + p1_ref[...]
        t = jnp.dot(t, wa_ref[...], preferred_element_type=jnp.float32) + ba_ref[...]
        t = jnp.maximum(t, 0.0)
        z = jnp.dot(t, wb_ref[...], preferred_element_type=jnp.float32) + bb_ref[...]
        m = jnp.max(z, axis=1, keepdims=True)
        e = z - m
        o_ref[...] = e - jnp.log(jnp.sum(jnp.exp(e), axis=1, keepdims=True))

    grid = (n // block_rows,)
    return pl.pallas_call(
        body,
        grid=grid,
        in_specs=[
            pl.BlockSpec(memory_space=pltpu.SMEM),
            pl.BlockSpec((block_rows, h), lambda i: (i, 0)),
            pl.BlockSpec((block_rows, h), lambda i: (i, 0)),
            pl.BlockSpec((block_rows, h), lambda i: (i, 0)),
            pl.BlockSpec(Wa.shape, lambda i: (0, 0)),
            pl.BlockSpec(ba.shape, lambda i: (0, 0)),
            pl.BlockSpec(Wb.shape, lambda i: (0, 0)),
            pl.BlockSpec(bb.shape, lambda i: (0, 0)),
        ],
        out_specs=pl.BlockSpec((block_rows, dout), lambda i: (i, 0)),
        out_shape=jax.ShapeDtypeStruct((n, dout), jnp.float32),
    )(eps, hin, p0, p1, Wa, ba, Wb, bb)


def kernel(x, edge_index, eps0, eps1, W0a, b0a, W0b, b0b, W1a, b1a, W1b, b1b):
    n = x.shape[0]
    e = edge_index.shape[1]
    src2 = edge_index[0].reshape(e // C, C)
    dst2 = edge_index[1].reshape(e // C, C)
    eps0_s = eps0.reshape(1, 1)
    eps1_s = eps1.reshape(1, 1)

    p = _segment_sum_sc(x, src2, dst2, n)
    h = _mlp0_tc(x, p[0], p[1], eps0_s, W0a, b0a.reshape(1, -1),
                 W0b, b0b.reshape(1, -1))
    q = _segment_sum_sc(h, src2, dst2, n)
    W1a_pad = jnp.concatenate([W1a, jnp.zeros_like(W1a)], axis=0)
    return _mlp1_tc(h, q[0], q[1], eps1_s, W1a_pad, b1a.reshape(1, -1),
                    W1b, b1b.reshape(1, -1))


# speedup vs baseline: 9.7496x; 1.3743x over previous
import functools

import jax
import jax.numpy as jnp
from jax import lax
from jax.experimental import pallas as pl
from jax.experimental.pallas import tpu as pltpu
from jax.experimental.pallas import tpu_sc as plsc

NC = 2
NS = 16
C = 80


def _segment_sum_sc(table, packed3, n_rows):
    _, nchunks, _ = packed3.shape
    d = table.shape[1]
    n_pad = ((n_rows + NS * 8 - 1) // (NS * 8)) * (NS * 8)
    rows_per_tile = n_pad // NS
    zr = 8
    mesh = plsc.VectorSubcoreMesh(core_axis_name="c", subcore_axis_name="s")

    @functools.partial(
        pl.kernel,
        out_type=jax.ShapeDtypeStruct((NC, n_pad, d), jnp.float32),
        mesh=mesh,
        scratch_types=[
            pltpu.VMEM((nchunks, C), jnp.int32),
            pltpu.VMEM((C,), jnp.int32),
            pltpu.VMEM((C,), jnp.int32),
            pltpu.VMEM((C,), jnp.int32),
            pltpu.VMEM((C,), jnp.int32),
            pltpu.VMEM((C, d), jnp.float32),
            pltpu.VMEM((C, d), jnp.float32),
            pltpu.VMEM((zr, d), jnp.float32),
            pltpu.VMEM_SHARED((n_pad, d), jnp.float32),
            pltpu.SemaphoreType.DMA,
            pltpu.SemaphoreType.DMA,
        ],
    )
    def seg_sum(table_hbm, idx_hbm, out_hbm,
                idx_v, srcb0, srcb1, dstb0, dstb1, rows0, rows1, zbuf, acc,
                sem0, sem1):
        cid = lax.axis_index("c")
        sid = lax.axis_index("s")
        tile = cid * NS + sid

        pltpu.sync_copy(idx_hbm.at[tile], idx_v)

        zero16 = jnp.zeros((16,), jnp.float32)

        def zrow(r, carry):
            for j in range(d // 16):
                zbuf[r, pl.ds(j * 16, 16)] = zero16
            return carry

        lax.fori_loop(0, zr, zrow, 0)
        row0 = sid * rows_per_tile
        for k in range(rows_per_tile // zr):
            pltpu.sync_copy(zbuf, acc.at[pl.ds(row0 + k * zr, zr)])
        plsc.subcore_barrier()

        def start(i, srcb, dstb, buf, sem):
            for k in range(C // 16):
                t = idx_v[i, pl.ds(k * 16, 16)]
                srcb[pl.ds(k * 16, 16)] = lax.shift_right_logical(t, 16)
                dstb[pl.ds(k * 16, 16)] = lax.bitwise_and(t, 0xFFFF)
            pltpu.async_copy(table_hbm.at[srcb], buf, sem)

        def finish(srcb, dstb, buf, sem):
            pltpu.make_async_copy(table_hbm.at[srcb], buf, sem).wait()
            pltpu.sync_copy(buf, acc.at[dstb], add=True)

        start(0, srcb0, dstb0, rows0, sem0)
        start(1, srcb1, dstb1, rows1, sem1)

        def body(j, carry):
            i = 2 * j
            finish(srcb0, dstb0, rows0, sem0)
            start(i + 2, srcb0, dstb0, rows0, sem0)
            finish(srcb1, dstb1, rows1, sem1)
            start(i + 3, srcb1, dstb1, rows1, sem1)
            return carry

        lax.fori_loop(0, (nchunks - 3) // 2, body, 0)
        finish(srcb0, dstb0, rows0, sem0)
        start(nchunks - 1, srcb0, dstb0, rows0, sem0)
        finish(srcb1, dstb1, rows1, sem1)
        finish(srcb0, dstb0, rows0, sem0)
        plsc.subcore_barrier()

        pltpu.sync_copy(acc.at[pl.ds(row0, rows_per_tile)],
                        out_hbm.at[cid, pl.ds(row0, rows_per_tile)])

    return seg_sum(table, packed3)[:, :n_rows, :]


def _mlp0_tc(x, p0, p1, eps, Wa, ba, Wb, bb, block_rows=1000):
    n, din = x.shape
    h = Wa.shape[1]

    def body(eps_ref, x_ref, p0_ref, p1_ref, wa_ref, ba_ref, wb_ref, bb_ref, o_ref):
        t = (1.0 + eps_ref[0, 0]) * x_ref[...] + p0_ref[...] + p1_ref[...]
        t = jnp.dot(t, wa_ref[...], preferred_element_type=jnp.float32) + ba_ref[...]
        t = jnp.maximum(t, 0.0)
        t = jnp.dot(t, wb_ref[...], preferred_element_type=jnp.float32) + bb_ref[...]
        t = jnp.maximum(t, 0.0)
        o_ref[...] = jnp.concatenate(
            [t, jnp.zeros((t.shape[0], t.shape[1]), jnp.float32)], axis=1)

    grid = (n // block_rows,)
    return pl.pallas_call(
        body,
        grid=grid,
        in_specs=[
            pl.BlockSpec(memory_space=pltpu.SMEM),
            pl.BlockSpec((block_rows, din), lambda i: (i, 0)),
            pl.BlockSpec((block_rows, din), lambda i: (i, 0)),
            pl.BlockSpec((block_rows, din), lambda i: (i, 0)),
            pl.BlockSpec(Wa.shape, lambda i: (0, 0)),
            pl.BlockSpec(ba.shape, lambda i: (0, 0)),
            pl.BlockSpec(Wb.shape, lambda i: (0, 0)),
            pl.BlockSpec(bb.shape, lambda i: (0, 0)),
        ],
        out_specs=pl.BlockSpec((block_rows, 2 * h), lambda i: (i, 0)),
        out_shape=jax.ShapeDtypeStruct((n, 2 * h), jnp.float32),
    )(eps, x, p0, p1, Wa, ba, Wb, bb)


def _mlp1_tc(hin, p0, p1, eps, Wa, ba, Wb, bb, block_rows=1000):
    n, h = hin.shape
    dout = Wb.shape[1]

    def body(eps_ref, h_ref, p0_ref, p1_ref, wa_ref, ba_ref, wb_ref, bb_ref, o_ref):
        t = (1.0 + eps_ref[0, 0]) * h_ref[...] + p0_ref[...] + p1_ref[...]
        t = jnp.dot(t, wa_ref[...], preferred_element_type=jnp.float32) + ba_ref[...]
        t = jnp.maximum(t, 0.0)
        z = jnp.dot(t, wb_ref[...], preferred_element_type=jnp.float32) + bb_ref[...]
        m = jnp.max(z, axis=1, keepdims=True)
        e = z - m
        o_ref[...] = e - jnp.log(jnp.sum(jnp.exp(e), axis=1, keepdims=True))

    grid = (n // block_rows,)
    return pl.pallas_call(
        body,
        grid=grid,
        in_specs=[
            pl.BlockSpec(memory_space=pltpu.SMEM),
            pl.BlockSpec((block_rows, h), lambda i: (i, 0)),
            pl.BlockSpec((block_rows, h), lambda i: (i, 0)),
            pl.BlockSpec((block_rows, h), lambda i: (i, 0)),
            pl.BlockSpec(Wa.shape, lambda i: (0, 0)),
            pl.BlockSpec(ba.shape, lambda i: (0, 0)),
            pl.BlockSpec(Wb.shape, lambda i: (0, 0)),
            pl.BlockSpec(bb.shape, lambda i: (0, 0)),
        ],
        out_specs=pl.BlockSpec((block_rows, dout), lambda i: (i, 0)),
        out_shape=jax.ShapeDtypeStruct((n, dout), jnp.float32),
    )(eps, hin, p0, p1, Wa, ba, Wb, bb)


def kernel(x, edge_index, eps0, eps1, W0a, b0a, W0b, b0b, W1a, b1a, W1b, b1b):
    n = x.shape[0]
    e = edge_index.shape[1]
    packed3 = (edge_index[0] * 65536 + edge_index[1]).reshape(
        NC * NS, e // (NC * NS * C), C)
    eps0_s = eps0.reshape(1, 1)
    eps1_s = eps1.reshape(1, 1)

    p = _segment_sum_sc(x, packed3, n)
    h = _mlp0_tc(x, p[0], p[1], eps0_s, W0a, b0a.reshape(1, -1),
                 W0b, b0b.reshape(1, -1))
    q = _segment_sum_sc(h, packed3, n)
    W1a_pad = jnp.concatenate([W1a, jnp.zeros_like(W1a)], axis=0)
    return _mlp1_tc(h, q[0], q[1], eps1_s, W1a_pad, b1a.reshape(1, -1),
                    W1b, b1b.reshape(1, -1))


# speedup vs baseline: 10.4008x; 1.0668x over previous
import functools

import jax
import jax.numpy as jnp
from jax import lax
from jax.experimental import pallas as pl
from jax.experimental.pallas import tpu as pltpu
from jax.experimental.pallas import tpu_sc as plsc

NC = 2
NS = 16
C = 80


def _segment_sum_sc(table, packed3, n_rows):
    _, nchunks, _ = packed3.shape
    d = table.shape[1]
    n_pad = ((n_rows + NS * 8 - 1) // (NS * 8)) * (NS * 8)
    rows_per_tile = n_pad // NS
    zr = 8
    mesh = plsc.VectorSubcoreMesh(core_axis_name="c", subcore_axis_name="s")

    @functools.partial(
        pl.kernel,
        out_type=jax.ShapeDtypeStruct((NC, n_pad, d), jnp.float32),
        mesh=mesh,
        scratch_types=[
            pltpu.VMEM((nchunks, C), jnp.int32),
            pltpu.VMEM((C,), jnp.int32),
            pltpu.VMEM((C,), jnp.int32),
            pltpu.VMEM((C,), jnp.int32),
            pltpu.VMEM((C,), jnp.int32),
            pltpu.VMEM((C, d), jnp.float32),
            pltpu.VMEM((C, d), jnp.float32),
            pltpu.VMEM((zr, d), jnp.float32),
            pltpu.VMEM_SHARED((n_pad, d), jnp.float32),
            pltpu.SemaphoreType.DMA,
            pltpu.SemaphoreType.DMA,
        ],
    )
    def seg_sum(table_hbm, idx_hbm, out_hbm,
                idx_v, srcb0, srcb1, dstb0, dstb1, rows0, rows1, zbuf, acc,
                sem0, sem1):
        cid = lax.axis_index("c")
        sid = lax.axis_index("s")
        tile = cid * NS + sid

        pltpu.sync_copy(idx_hbm.at[tile], idx_v)

        zero16 = jnp.zeros((16,), jnp.float32)

        def zrow(r, carry):
            for j in range(d // 16):
                zbuf[r, pl.ds(j * 16, 16)] = zero16
            return carry

        lax.fori_loop(0, zr, zrow, 0)
        row0 = sid * rows_per_tile
        for k in range(rows_per_tile // zr):
            pltpu.sync_copy(zbuf, acc.at[pl.ds(row0 + k * zr, zr)])
        plsc.subcore_barrier()

        def start(i, srcb, dstb, buf, sem):
            for k in range(C // 16):
                t = idx_v[i, pl.ds(k * 16, 16)]
                srcb[pl.ds(k * 16, 16)] = lax.shift_right_logical(t, 16)
                dstb[pl.ds(k * 16, 16)] = lax.bitwise_and(t, 0xFFFF)
            pltpu.async_copy(table_hbm.at[srcb], buf, sem)

        def finish(srcb, dstb, buf, sem):
            pltpu.make_async_copy(table_hbm.at[srcb], buf, sem).wait()
            pltpu.sync_copy(buf, acc.at[dstb], add=True)

        start(0, srcb0, dstb0, rows0, sem0)
        start(1, srcb1, dstb1, rows1, sem1)

        def body(j, carry):
            i = 2 * j
            finish(srcb0, dstb0, rows0, sem0)
            start(i + 2, srcb0, dstb0, rows0, sem0)
            finish(srcb1, dstb1, rows1, sem1)
            start(i + 3, srcb1, dstb1, rows1, sem1)
            return carry

        lax.fori_loop(0, (nchunks - 3) // 2, body, 0)
        finish(srcb0, dstb0, rows0, sem0)
        start(nchunks - 1, srcb0, dstb0, rows0, sem0)
        finish(srcb1, dstb1, rows1, sem1)
        finish(srcb0, dstb0, rows0, sem0)
        plsc.subcore_barrier()

        pltpu.sync_copy(acc.at[pl.ds(row0, rows_per_tile)],
                        out_hbm.at[cid, pl.ds(row0, rows_per_tile)])

    return seg_sum(table, packed3)


def _mlp0_tc(x, p, eps, Wa, ba, Wb, bb, block_rows=2000):
    n, din = x.shape
    h = Wa.shape[1]

    def body(eps_ref, x_ref, p_ref, wa_ref, ba_ref, wb_ref, bb_ref, o_ref):
        t = (1.0 + eps_ref[0, 0]) * x_ref[...] + p_ref[0] + p_ref[1]
        t = jnp.dot(t, wa_ref[...], preferred_element_type=jnp.float32) + ba_ref[...]
        t = jnp.maximum(t, 0.0)
        t = jnp.dot(t, wb_ref[...], preferred_element_type=jnp.float32) + bb_ref[...]
        t = jnp.maximum(t, 0.0)
        o_ref[...] = jnp.concatenate(
            [t, jnp.zeros((t.shape[0], t.shape[1]), jnp.float32)], axis=1)

    grid = (n // block_rows,)
    return pl.pallas_call(
        body,
        grid=grid,
        in_specs=[
            pl.BlockSpec(memory_space=pltpu.SMEM),
            pl.BlockSpec((block_rows, din), lambda i: (i, 0)),
            pl.BlockSpec((2, block_rows, din), lambda i: (0, i, 0)),
            pl.BlockSpec(Wa.shape, lambda i: (0, 0)),
            pl.BlockSpec(ba.shape, lambda i: (0, 0)),
            pl.BlockSpec(Wb.shape, lambda i: (0, 0)),
            pl.BlockSpec(bb.shape, lambda i: (0, 0)),
        ],
        out_specs=pl.BlockSpec((block_rows, 2 * h), lambda i: (i, 0)),
        out_shape=jax.ShapeDtypeStruct((n, 2 * h), jnp.float32),
    )(eps, x, p, Wa, ba, Wb, bb)


def _mlp1_tc(hin, q, eps, Wa, ba, Wb, bb, block_rows=2000):
    n, h = hin.shape
    dout = Wb.shape[1]

    def body(eps_ref, h_ref, q_ref, wa_ref, ba_ref, wb_ref, bb_ref, o_ref):
        t = (1.0 + eps_ref[0, 0]) * h_ref[...] + q_ref[0] + q_ref[1]
        t = jnp.dot(t, wa_ref[...], preferred_element_type=jnp.float32) + ba_ref[...]
        t = jnp.maximum(t, 0.0)
        z = jnp.dot(t, wb_ref[...], preferred_element_type=jnp.float32) + bb_ref[...]
        m = jnp.max(z, axis=1, keepdims=True)
        e = z - m
        o_ref[...] = e - jnp.log(jnp.sum(jnp.exp(e), axis=1, keepdims=True))

    grid = (n // block_rows,)
    return pl.pallas_call(
        body,
        grid=grid,
        in_specs=[
            pl.BlockSpec(memory_space=pltpu.SMEM),
            pl.BlockSpec((block_rows, h), lambda i: (i, 0)),
            pl.BlockSpec((2, block_rows, h), lambda i: (0, i, 0)),
            pl.BlockSpec(Wa.shape, lambda i: (0, 0)),
            pl.BlockSpec(ba.shape, lambda i: (0, 0)),
            pl.BlockSpec(Wb.shape, lambda i: (0, 0)),
            pl.BlockSpec(bb.shape, lambda i: (0, 0)),
        ],
        out_specs=pl.BlockSpec((block_rows, dout), lambda i: (i, 0)),
        out_shape=jax.ShapeDtypeStruct((n, dout), jnp.float32),
    )(eps, hin, q, Wa, ba, Wb, bb)


def kernel(x, edge_index, eps0, eps1, W0a, b0a, W0b, b0b, W1a, b1a, W1b, b1b):
    n = x.shape[0]
    e = edge_index.shape[1]
    packed3 = (edge_index[0] * 65536 + edge_index[1]).reshape(
        NC * NS, e // (NC * NS * C), C)
    eps0_s = eps0.reshape(1, 1)
    eps1_s = eps1.reshape(1, 1)

    p = _segment_sum_sc(x, packed3, n)
    h = _mlp0_tc(x, p, eps0_s, W0a, b0a.reshape(1, -1),
                 W0b, b0b.reshape(1, -1))
    q = _segment_sum_sc(h, packed3, n)
    W1a_pad = jnp.concatenate([W1a, jnp.zeros_like(W1a)], axis=0)
    return _mlp1_tc(h, q, eps1_s, W1a_pad, b1a.reshape(1, -1),
                    W1b, b1b.reshape(1, -1))


# speedup vs baseline: 11.3474x; 1.0910x over previous
import functools

import jax
import jax.numpy as jnp
from jax import lax
from jax.experimental import pallas as pl
from jax.experimental.pallas import tpu as pltpu
from jax.experimental.pallas import tpu_sc as plsc

NC = 2
NS = 16
C = 80


def _segment_sum_sc(table, packed3, n_rows):
    _, nchunks, _ = packed3.shape
    d = table.shape[1]
    n_pad = ((n_rows + NS * 8 - 1) // (NS * 8)) * (NS * 8)
    rows_per_tile = n_pad // NS
    zr = 8
    mesh = plsc.VectorSubcoreMesh(core_axis_name="c", subcore_axis_name="s")

    @functools.partial(
        pl.kernel,
        out_type=jax.ShapeDtypeStruct((NC, n_pad, d), jnp.float32),
        mesh=mesh,
        compiler_params=pltpu.CompilerParams(use_tc_tiling_on_sc=False),
        scratch_types=[
            pltpu.VMEM((nchunks, C), jnp.int32),
            pltpu.VMEM((C,), jnp.int32),
            pltpu.VMEM((C,), jnp.int32),
            pltpu.VMEM((C,), jnp.int32),
            pltpu.VMEM((C,), jnp.int32),
            pltpu.VMEM((C, d), jnp.float32),
            pltpu.VMEM((C, d), jnp.float32),
            pltpu.VMEM((zr, d), jnp.float32),
            pltpu.VMEM_SHARED((n_pad, d), jnp.float32),
            pltpu.SemaphoreType.DMA,
            pltpu.SemaphoreType.DMA,
        ],
    )
    def seg_sum(table_hbm, idx_hbm, out_hbm,
                idx_v, srcb0, srcb1, dstb0, dstb1, rows0, rows1, zbuf, acc,
                sem0, sem1):
        cid = lax.axis_index("c")
        sid = lax.axis_index("s")
        tile = cid * NS + sid

        pltpu.sync_copy(idx_hbm.at[tile], idx_v)

        zero16 = jnp.zeros((16,), jnp.float32)

        def zrow(r, carry):
            for j in range(d // 16):
                zbuf[r, pl.ds(j * 16, 16)] = zero16
            return carry

        lax.fori_loop(0, zr, zrow, 0)
        row0 = sid * rows_per_tile
        for k in range(rows_per_tile // zr):
            pltpu.sync_copy(zbuf, acc.at[pl.ds(row0 + k * zr, zr)])
        plsc.subcore_barrier()

        def start(i, srcb, dstb, buf, sem):
            for k in range(C // 16):
                t = idx_v[i, pl.ds(k * 16, 16)]
                srcb[pl.ds(k * 16, 16)] = lax.shift_right_logical(t, 16)
                dstb[pl.ds(k * 16, 16)] = lax.bitwise_and(t, 0xFFFF)
            pltpu.async_copy(table_hbm.at[srcb], buf, sem)

        def finish(srcb, dstb, buf, sem):
            pltpu.make_async_copy(table_hbm.at[srcb], buf, sem).wait()
            pltpu.sync_copy(buf, acc.at[dstb], add=True)

        start(0, srcb0, dstb0, rows0, sem0)
        start(1, srcb1, dstb1, rows1, sem1)

        def body(j, carry):
            i = 2 * j
            finish(srcb0, dstb0, rows0, sem0)
            start(i + 2, srcb0, dstb0, rows0, sem0)
            finish(srcb1, dstb1, rows1, sem1)
            start(i + 3, srcb1, dstb1, rows1, sem1)
            return carry

        lax.fori_loop(0, (nchunks - 3) // 2, body, 0)
        finish(srcb0, dstb0, rows0, sem0)
        start(nchunks - 1, srcb0, dstb0, rows0, sem0)
        finish(srcb1, dstb1, rows1, sem1)
        finish(srcb0, dstb0, rows0, sem0)
        plsc.subcore_barrier()

        pltpu.sync_copy(acc.at[pl.ds(row0, rows_per_tile)],
                        out_hbm.at[cid, pl.ds(row0, rows_per_tile)])

    return seg_sum(table, packed3)


def _mlp0_tc(x, p, eps, Wa, ba, Wb, bb, block_rows=2000):
    n, din = x.shape
    h = Wa.shape[1]

    def body(eps_ref, x_ref, p_ref, wa_ref, ba_ref, wb_ref, bb_ref, o_ref):
        t = (1.0 + eps_ref[0, 0]) * x_ref[...] + p_ref[0] + p_ref[1]
        t = jnp.dot(t, wa_ref[...], preferred_element_type=jnp.float32) + ba_ref[...]
        t = jnp.maximum(t, 0.0)
        t = jnp.dot(t, wb_ref[...], preferred_element_type=jnp.float32) + bb_ref[...]
        o_ref[...] = jnp.maximum(t, 0.0)

    grid = (n // block_rows,)
    return pl.pallas_call(
        body,
        grid=grid,
        in_specs=[
            pl.BlockSpec(memory_space=pltpu.SMEM),
            pl.BlockSpec((block_rows, din), lambda i: (i, 0)),
            pl.BlockSpec((2, block_rows, din), lambda i: (0, i, 0)),
            pl.BlockSpec(Wa.shape, lambda i: (0, 0)),
            pl.BlockSpec(ba.shape, lambda i: (0, 0)),
            pl.BlockSpec(Wb.shape, lambda i: (0, 0)),
            pl.BlockSpec(bb.shape, lambda i: (0, 0)),
        ],
        out_specs=pl.BlockSpec((block_rows, h), lambda i: (i, 0)),
        out_shape=jax.ShapeDtypeStruct((n, h), jnp.float32),
    )(eps, x, p, Wa, ba, Wb, bb)


def _mlp1_tc(hin, q, eps, Wa, ba, Wb, bb, block_rows=2000):
    n, h = hin.shape
    dout = Wb.shape[1]

    def body(eps_ref, h_ref, q_ref, wa_ref, ba_ref, wb_ref, bb_ref, o_ref):
        t = (1.0 + eps_ref[0, 0]) * h_ref[...] + q_ref[0] + q_ref[1]
        t = jnp.dot(t, wa_ref[...], preferred_element_type=jnp.float32) + ba_ref[...]
        t = jnp.maximum(t, 0.0)
        z = jnp.dot(t, wb_ref[...], preferred_element_type=jnp.float32) + bb_ref[...]
        m = jnp.max(z, axis=1, keepdims=True)
        e = z - m
        o_ref[...] = e - jnp.log(jnp.sum(jnp.exp(e), axis=1, keepdims=True))

    grid = (n // block_rows,)
    return pl.pallas_call(
        body,
        grid=grid,
        in_specs=[
            pl.BlockSpec(memory_space=pltpu.SMEM),
            pl.BlockSpec((block_rows, h), lambda i: (i, 0)),
            pl.BlockSpec((2, block_rows, h), lambda i: (0, i, 0)),
            pl.BlockSpec(Wa.shape, lambda i: (0, 0)),
            pl.BlockSpec(ba.shape, lambda i: (0, 0)),
            pl.BlockSpec(Wb.shape, lambda i: (0, 0)),
            pl.BlockSpec(bb.shape, lambda i: (0, 0)),
        ],
        out_specs=pl.BlockSpec((block_rows, dout), lambda i: (i, 0)),
        out_shape=jax.ShapeDtypeStruct((n, dout), jnp.float32),
    )(eps, hin, q, Wa, ba, Wb, bb)


def kernel(x, edge_index, eps0, eps1, W0a, b0a, W0b, b0b, W1a, b1a, W1b, b1b):
    n = x.shape[0]
    e = edge_index.shape[1]
    packed3 = (edge_index[0] * 65536 + edge_index[1]).reshape(
        NC * NS, e // (NC * NS * C), C)
    eps0_s = eps0.reshape(1, 1)
    eps1_s = eps1.reshape(1, 1)

    p = _segment_sum_sc(x, packed3, n)
    h = _mlp0_tc(x, p, eps0_s, W0a, b0a.reshape(1, -1),
                 W0b, b0b.reshape(1, -1))
    q = _segment_sum_sc(h, packed3, n)
    return _mlp1_tc(h, q, eps1_s, W1a, b1a.reshape(1, -1),
                    W1b, b1b.reshape(1, -1))
